# baseline (speedup 1.0000x reference)
import functools

import jax
import jax.numpy as jnp
from jax import lax
from jax.experimental import pallas as pl
from jax.experimental.pallas import tpu as pltpu

NUM_EMB = 1024
DIM = 256
N_TOK = 16 * 1024
CCOST = 0.25


def _vq_body(x2_ref, x_ref, w_ref, w2_ref,
             qst_ref, idx_ref, counts_ref, loss_ref, perp_ref):
    i = pl.program_id(0)
    g = pl.num_programs(0)
    x = x_ref[...]
    w = w_ref[...]

    scores = lax.dot_general(
        x, w, (((1,), (1,)), ((), ())),
        preferred_element_type=jnp.float32)
    t = x2_ref[...] + w2_ref[...]
    d = t - 2.0 * scores

    m = jnp.min(d, axis=1, keepdims=True)
    lane = lax.broadcasted_iota(jnp.int32, d.shape, 1)
    idx = jnp.min(jnp.where(d == m, lane, NUM_EMB), axis=1, keepdims=True)
    idx_ref[...] = idx.reshape(1, 1, -1)

    onehot = (lane == idx).astype(jnp.float32)
    q = lax.dot_general(
        onehot, w, (((1,), (0,)), ((), ())),
        preferred_element_type=jnp.float32)
    qst_ref[...] = x + (q - x)

    @pl.when(i == 0)
    def _init():
        counts_ref[...] = jnp.zeros_like(counts_ref)
        loss_ref[...] = jnp.zeros_like(loss_ref)

    counts_ref[...] += jnp.sum(onehot, axis=0, keepdims=True)
    loss_ref[...] += jnp.sum(m).reshape(1, 1)

    @pl.when(i == g - 1)
    def _finalize():
        s = loss_ref[...] / jnp.float32(N_TOK * DIM)
        loss_ref[...] = s + CCOST * s
        p = counts_ref[...] / jnp.float32(N_TOK)
        perp_ref[...] = jnp.exp(-jnp.sum(p * jnp.log(p + 1e-10))).reshape(1, 1)


@functools.partial(jax.jit, static_argnames=("block",))
def _vq_tc(flat, x2, w, w2, block=512):
    g = N_TOK // block
    out = pl.pallas_call(
        _vq_body,
        grid=(g,),
        in_specs=[
            pl.BlockSpec((block, 1), lambda i: (i, 0)),
            pl.BlockSpec((block, DIM), lambda i: (i, 0)),
            pl.BlockSpec((NUM_EMB, DIM), lambda i: (0, 0)),
            pl.BlockSpec((1, NUM_EMB), lambda i: (0, 0)),
        ],
        out_specs=[
            pl.BlockSpec((block, DIM), lambda i: (i, 0)),
            pl.BlockSpec((1, 1, block), lambda i: (i, 0, 0)),
            pl.BlockSpec((1, NUM_EMB), lambda i: (0, 0)),
            pl.BlockSpec((1, 1), lambda i: (0, 0)),
            pl.BlockSpec((1, 1), lambda i: (0, 0)),
        ],
        out_shape=[
            jax.ShapeDtypeStruct((N_TOK, DIM), jnp.float32),
            jax.ShapeDtypeStruct((g, 1, block), jnp.int32),
            jax.ShapeDtypeStruct((1, NUM_EMB), jnp.float32),
            jax.ShapeDtypeStruct((1, 1), jnp.float32),
            jax.ShapeDtypeStruct((1, 1), jnp.float32),
        ],
        compiler_params=pltpu.CompilerParams(
            dimension_semantics=("arbitrary",)),
    )(x2, flat, w, w2)
    return out


def kernel(inputs, embedding_weight):
    input_shape = inputs.shape
    flat = inputs.reshape(-1, DIM)
    x2 = jnp.sum(flat ** 2, axis=1, keepdims=True)
    w2 = jnp.sum(embedding_weight ** 2, axis=1)[None, :]
    qst, idx, _counts, loss, perp = _vq_tc(flat, x2, embedding_weight, w2)
    return (qst.reshape(input_shape),
            loss.reshape(()),
            perp.reshape(()),
            idx.reshape(input_shape[:-1]))


# speedup vs baseline: 1.0356x; 1.0356x over previous
import functools

import jax
import jax.numpy as jnp
from jax import lax
from jax.experimental import pallas as pl
from jax.experimental.pallas import tpu as pltpu

NUM_EMB = 1024
DIM = 256
N_TOK = 16 * 1024
CCOST = 0.25


def _vq_body(x2_ref, x_ref, w_ref, w2_ref, lane_ref,
             qst_ref, idx_ref, counts_ref, loss_ref, perp_ref):
    i = pl.program_id(0)
    g = pl.num_programs(0)
    x = x_ref[...]
    w = w_ref[...]
    lane = lane_ref[...]

    scores = lax.dot_general(
        x, w, (((1,), (1,)), ((), ())),
        preferred_element_type=jnp.float32)
    t = x2_ref[...] + w2_ref[...]
    d = t - 2.0 * scores

    m = jnp.min(d, axis=1, keepdims=True)
    idxf = jnp.min(jnp.where(d == m, lane, jnp.float32(NUM_EMB)),
                   axis=1, keepdims=True)
    idx_ref[...] = idxf.astype(jnp.int32).reshape(1, 1, -1)

    onehot = (lane == idxf).astype(jnp.float32)
    q = lax.dot_general(
        onehot, w, (((1,), (0,)), ((), ())),
        preferred_element_type=jnp.float32)
    qst_ref[...] = x + (q - x)

    @pl.when(i == 0)
    def _init():
        counts_ref[...] = jnp.zeros_like(counts_ref)
        loss_ref[...] = jnp.zeros_like(loss_ref)

    counts_ref[...] += jnp.sum(onehot, axis=0, keepdims=True)
    loss_ref[...] += jnp.sum(m).reshape(1, 1)

    @pl.when(i == g - 1)
    def _finalize():
        s = loss_ref[...] / jnp.float32(N_TOK * DIM)
        loss_ref[...] = s + CCOST * s
        p = counts_ref[...] / jnp.float32(N_TOK)
        perp_ref[...] = jnp.exp(-jnp.sum(p * jnp.log(p + 1e-10))).reshape(1, 1)


@functools.partial(jax.jit, static_argnames=("block",))
def _vq_tc(flat, x2, w, w2, lane, block=512):
    g = N_TOK // block
    out = pl.pallas_call(
        _vq_body,
        grid=(g,),
        in_specs=[
            pl.BlockSpec((block, 1), lambda i: (i, 0)),
            pl.BlockSpec((block, DIM), lambda i: (i, 0)),
            pl.BlockSpec((NUM_EMB, DIM), lambda i: (0, 0)),
            pl.BlockSpec((1, NUM_EMB), lambda i: (0, 0)),
            pl.BlockSpec((1, NUM_EMB), lambda i: (0, 0)),
        ],
        out_specs=[
            pl.BlockSpec((block, DIM), lambda i: (i, 0)),
            pl.BlockSpec((1, 1, block), lambda i: (i, 0, 0)),
            pl.BlockSpec((1, NUM_EMB), lambda i: (0, 0)),
            pl.BlockSpec((1, 1), lambda i: (0, 0)),
            pl.BlockSpec((1, 1), lambda i: (0, 0)),
        ],
        out_shape=[
            jax.ShapeDtypeStruct((N_TOK, DIM), jnp.float32),
            jax.ShapeDtypeStruct((g, 1, block), jnp.int32),
            jax.ShapeDtypeStruct((1, NUM_EMB), jnp.float32),
            jax.ShapeDtypeStruct((1, 1), jnp.float32),
            jax.ShapeDtypeStruct((1, 1), jnp.float32),
        ],
        compiler_params=pltpu.CompilerParams(
            dimension_semantics=("arbitrary",)),
    )(x2, flat, w, w2, lane)
    return out


def kernel(inputs, embedding_weight):
    input_shape = inputs.shape
    flat = inputs.reshape(-1, DIM)
    x2 = jnp.sum(flat ** 2, axis=1, keepdims=True)
    w2 = jnp.sum(embedding_weight ** 2, axis=1)[None, :]
    lane = lax.broadcasted_iota(jnp.float32, (1, NUM_EMB), 1)
    qst, idx, _counts, loss, perp = _vq_tc(flat, x2, embedding_weight, w2, lane)
    return (qst.reshape(input_shape),
            loss.reshape(()),
            perp.reshape(()),
            idx.reshape(input_shape[:-1]))


# speedup vs baseline: 1.1695x; 1.1293x over previous
import functools

import jax
import jax.numpy as jnp
from jax import lax
from jax.experimental import pallas as pl
from jax.experimental.pallas import tpu as pltpu

NUM_EMB = 1024
DIM = 256
N_TOK = 16 * 1024
CCOST = 0.25


def _vq_body(x2_ref, x_ref, w_ref, w2_ref, lane_ref,
             qst_ref, idx_ref, counts_ref, loss_ref, perp_ref):
    i = pl.program_id(0)
    g = pl.num_programs(0)
    x = x_ref[...]
    w = w_ref[...]
    lane = lane_ref[...]

    scores = lax.dot_general(
        x, w, (((1,), (1,)), ((), ())),
        preferred_element_type=jnp.float32)
    t = x2_ref[...] + w2_ref[...]
    d = t - 2.0 * scores

    m = jnp.min(d, axis=1, keepdims=True)
    idxf = jnp.min(jnp.where(d == m, lane, jnp.float32(NUM_EMB)),
                   axis=1, keepdims=True)
    idx_ref[...] = idxf.astype(jnp.int32).reshape(1, 1, -1)

    onehot = (lane == idxf).astype(jnp.float32)
    q = lax.dot_general(
        onehot, w, (((1,), (0,)), ((), ())),
        preferred_element_type=jnp.float32)
    qst_ref[...] = x + (q - x)

    @pl.when(i == 0)
    def _init():
        counts_ref[...] = jnp.zeros_like(counts_ref)
        loss_ref[...] = jnp.zeros_like(loss_ref)

    counts_ref[...] += jnp.sum(onehot, axis=0, keepdims=True)
    loss_ref[...] += jnp.sum(m).reshape(1, 1)

    @pl.when(i == g - 1)
    def _finalize():
        s = loss_ref[...] / jnp.float32(N_TOK * DIM)
        loss_ref[...] = s + CCOST * s
        p = counts_ref[...] / jnp.float32(N_TOK)
        perp_ref[...] = jnp.exp(-jnp.sum(p * jnp.log(p + 1e-10))).reshape(1, 1)


@functools.partial(jax.jit, static_argnames=("block",))
def _vq_tc(flat, x2, w, w2, lane, block=1024):
    g = N_TOK // block
    out = pl.pallas_call(
        _vq_body,
        grid=(g,),
        in_specs=[
            pl.BlockSpec((block, 1), lambda i: (i, 0)),
            pl.BlockSpec((block, DIM), lambda i: (i, 0)),
            pl.BlockSpec((NUM_EMB, DIM), lambda i: (0, 0)),
            pl.BlockSpec((1, NUM_EMB), lambda i: (0, 0)),
            pl.BlockSpec((1, NUM_EMB), lambda i: (0, 0)),
        ],
        out_specs=[
            pl.BlockSpec((block, DIM), lambda i: (i, 0)),
            pl.BlockSpec((1, 1, block), lambda i: (i, 0, 0)),
            pl.BlockSpec((1, NUM_EMB), lambda i: (0, 0)),
            pl.BlockSpec((1, 1), lambda i: (0, 0)),
            pl.BlockSpec((1, 1), lambda i: (0, 0)),
        ],
        out_shape=[
            jax.ShapeDtypeStruct((N_TOK, DIM), jnp.float32),
            jax.ShapeDtypeStruct((g, 1, block), jnp.int32),
            jax.ShapeDtypeStruct((1, NUM_EMB), jnp.float32),
            jax.ShapeDtypeStruct((1, 1), jnp.float32),
            jax.ShapeDtypeStruct((1, 1), jnp.float32),
        ],
        compiler_params=pltpu.CompilerParams(
            dimension_semantics=("arbitrary",)),
    )(x2, flat, w, w2, lane)
    return out


def kernel(inputs, embedding_weight):
    input_shape = inputs.shape
    flat = inputs.reshape(-1, DIM)
    x2 = jnp.sum(flat ** 2, axis=1, keepdims=True)
    w2 = jnp.sum(embedding_weight ** 2, axis=1)[None, :]
    lane = lax.broadcasted_iota(jnp.float32, (1, NUM_EMB), 1)
    qst, idx, _counts, loss, perp = _vq_tc(flat, x2, embedding_weight, w2, lane)
    return (qst.reshape(input_shape),
            loss.reshape(()),
            perp.reshape(()),
            idx.reshape(input_shape[:-1]))


# speedup vs baseline: 1.2198x; 1.0430x over previous
import functools

import jax
import jax.numpy as jnp
from jax import lax
from jax.experimental import pallas as pl
from jax.experimental.pallas import tpu as pltpu

NUM_EMB = 1024
DIM = 256
N_TOK = 16 * 1024
CCOST = 0.25


def _vq_body(x2_ref, x_ref, w_ref, w2_ref, lane_ref,
             qst_ref, idx_ref, counts_ref, loss_ref, perp_ref):
    i = pl.program_id(0)
    g = pl.num_programs(0)
    x = x_ref[...]
    w = w_ref[...]
    lane = lane_ref[...]

    scores = lax.dot_general(
        x, w, (((1,), (1,)), ((), ())),
        preferred_element_type=jnp.float32)
    t = x2_ref[...] + w2_ref[...]
    d = t - 2.0 * scores

    m = jnp.min(d, axis=1, keepdims=True)
    idxf = jnp.min(jnp.where(d == m, lane, jnp.float32(NUM_EMB)),
                   axis=1, keepdims=True)
    idx_ref[...] = idxf.astype(jnp.int32).reshape(1, 1, -1)

    onehot = (lane == idxf).astype(jnp.float32)
    q = lax.dot_general(
        onehot, w, (((1,), (0,)), ((), ())),
        preferred_element_type=jnp.float32)
    qst_ref[...] = x + (q - x)

    @pl.when(i == 0)
    def _init():
        counts_ref[...] = jnp.zeros_like(counts_ref)
        loss_ref[...] = jnp.zeros_like(loss_ref)

    counts_ref[...] += jnp.sum(onehot, axis=0, keepdims=True)
    loss_ref[...] += jnp.sum(m).reshape(1, 1)

    @pl.when(i == g - 1)
    def _finalize():
        s = loss_ref[...] / jnp.float32(N_TOK * DIM)
        loss_ref[...] = s + CCOST * s
        p = counts_ref[...] / jnp.float32(N_TOK)
        perp_ref[...] = jnp.exp(-jnp.sum(p * jnp.log(p + 1e-10))).reshape(1, 1)


@functools.partial(jax.jit, static_argnames=("block",))
def _vq_tc(flat, x2, w, w2, lane, block=2048):
    g = N_TOK // block
    out = pl.pallas_call(
        _vq_body,
        grid=(g,),
        in_specs=[
            pl.BlockSpec((block, 1), lambda i: (i, 0)),
            pl.BlockSpec((block, DIM), lambda i: (i, 0)),
            pl.BlockSpec((NUM_EMB, DIM), lambda i: (0, 0)),
            pl.BlockSpec((1, NUM_EMB), lambda i: (0, 0)),
            pl.BlockSpec((1, NUM_EMB), lambda i: (0, 0)),
        ],
        out_specs=[
            pl.BlockSpec((block, DIM), lambda i: (i, 0)),
            pl.BlockSpec((1, 1, block), lambda i: (i, 0, 0)),
            pl.BlockSpec((1, NUM_EMB), lambda i: (0, 0)),
            pl.BlockSpec((1, 1), lambda i: (0, 0)),
            pl.BlockSpec((1, 1), lambda i: (0, 0)),
        ],
        out_shape=[
            jax.ShapeDtypeStruct((N_TOK, DIM), jnp.float32),
            jax.ShapeDtypeStruct((g, 1, block), jnp.int32),
            jax.ShapeDtypeStruct((1, NUM_EMB), jnp.float32),
            jax.ShapeDtypeStruct((1, 1), jnp.float32),
            jax.ShapeDtypeStruct((1, 1), jnp.float32),
        ],
        compiler_params=pltpu.CompilerParams(
            dimension_semantics=("arbitrary",)),
    )(x2, flat, w, w2, lane)
    return out


def kernel(inputs, embedding_weight):
    input_shape = inputs.shape
    flat = inputs.reshape(-1, DIM)
    x2 = jnp.sum(flat ** 2, axis=1, keepdims=True)
    w2 = jnp.sum(embedding_weight ** 2, axis=1)[None, :]
    lane = lax.broadcasted_iota(jnp.float32, (1, NUM_EMB), 1)
    qst, idx, _counts, loss, perp = _vq_tc(flat, x2, embedding_weight, w2, lane)
    return (qst.reshape(input_shape),
            loss.reshape(()),
            perp.reshape(()),
            idx.reshape(input_shape[:-1]))
